# baseline (speedup 1.0000x reference)
import functools

import jax
import jax.numpy as jnp
from jax import lax
from jax.experimental import pallas as pl
from jax.experimental.pallas import tpu as pltpu
from jax.experimental.pallas import tpu_sc as plsc

B, D, A, S = 16384, 128, 64, 64
BLK = 8192

NC, NS, L = 2, 16, 16
NW = NC * NS
W_ELEMS = B * A // NW


@functools.partial(
    pl.kernel,
    out_type=jax.ShapeDtypeStruct((B * A,), jnp.float32),
    mesh=plsc.VectorSubcoreMesh(core_axis_name="c", subcore_axis_name="s"),
    scratch_types=[pltpu.VMEM((W_ELEMS,), jnp.float32)],
    compiler_params=pltpu.CompilerParams(needs_layout_passes=False),
)
def _std_sc(out_hbm, buf_v):
    wid = lax.axis_index("s") * NC + lax.axis_index("c")
    zeros16 = jnp.zeros((L,), jnp.float32)

    def zero_body(i, carry):
        b = i * (8 * L)
        for k in range(8):
            buf_v[pl.ds(b + k * L, L)] = zeros16
        return carry

    lax.fori_loop(0, W_ELEMS // (8 * L), zero_body, 0)
    pltpu.sync_copy(buf_v, out_hbm.at[pl.ds(wid * W_ELEMS, W_ELEMS)])


def _tc_body(state_ref, wt_ref, bias_ref, wv_ref, bv_ref,
             mean_ref, value_ref, onehot_ref):
    x = state_ref[...]
    res = jnp.dot(x, wt_ref[...]) + bias_ref[...]
    mean_ref[...] = res[:, :A]
    value_ref[...] = lax.dot_general(
        wv_ref[...], x, (((1,), (1,)), ((), ()))) + bv_ref[...]
    logits = res[:, A:]
    idx = jnp.argmax(logits, axis=1)
    onehot_ref[...] = (
        lax.broadcasted_iota(jnp.int32, (BLK, S), 1) == idx[:, None]
    ).astype(jnp.float32)


@jax.jit
def kernel(state, W_skill, b_skill, W_action, b_action, W_value, b_value):
    wt = jnp.concatenate([W_action.T, W_skill.T], axis=1)
    bias = jnp.concatenate([b_action, b_skill])[None, :]

    grid = (B // BLK,)
    mean, value, one_hot = pl.pallas_call(
        _tc_body,
        grid=grid,
        in_specs=[
            pl.BlockSpec((BLK, D), lambda i: (i, 0)),
            pl.BlockSpec((D, 128), lambda i: (0, 0)),
            pl.BlockSpec((1, 128), lambda i: (0, 0)),
            pl.BlockSpec((1, D), lambda i: (0, 0)),
            pl.BlockSpec((1, 1), lambda i: (0, 0)),
        ],
        out_specs=[
            pl.BlockSpec((BLK, A), lambda i: (i, 0)),
            pl.BlockSpec((1, BLK), lambda i: (0, i)),
            pl.BlockSpec((BLK, S), lambda i: (i, 0)),
        ],
        out_shape=[
            jax.ShapeDtypeStruct((B, A), jnp.float32),
            jax.ShapeDtypeStruct((1, B), jnp.float32),
            jax.ShapeDtypeStruct((B, S), jnp.float32),
        ],
        compiler_params=pltpu.CompilerParams(
            dimension_semantics=("arbitrary",),
        ),
    )(state, wt, bias, W_value, b_value[None, :])
    std = _std_sc().reshape(B, A)
    return (mean, std, value[0], one_hot)


# speedup vs baseline: 1.7471x; 1.7471x over previous
import functools

import jax
import jax.numpy as jnp
from jax import lax
from jax.experimental import pallas as pl
from jax.experimental.pallas import tpu as pltpu

B, D, A, S = 16384, 128, 64, 64
BLK = 16384


def _tc_body(state_ref, wt_ref, bias_ref, wv_ref, bv_ref,
             mean_ref, value_ref, onehot_ref):
    x = state_ref[...]
    res = jnp.dot(x, wt_ref[...]) + bias_ref[...]
    mean_ref[...] = res[:, :A]
    value_ref[...] = lax.dot_general(
        wv_ref[...], x, (((1,), (1,)), ((), ()))) + bv_ref[...]
    logits = res[:, A:]
    idx = jnp.argmax(logits, axis=1)
    onehot_ref[...] = (
        lax.broadcasted_iota(jnp.int32, (BLK, S), 1) == idx[:, None]
    ).astype(jnp.float32)


@jax.jit
def kernel(state, W_skill, b_skill, W_action, b_action, W_value, b_value):
    wt = jnp.concatenate([W_action.T, W_skill.T], axis=1)
    bias = jnp.concatenate([b_action, b_skill])[None, :]

    grid = (B // BLK,)
    mean, value, one_hot = pl.pallas_call(
        _tc_body,
        grid=grid,
        in_specs=[
            pl.BlockSpec((BLK, D), lambda i: (i, 0)),
            pl.BlockSpec((D, 128), lambda i: (0, 0)),
            pl.BlockSpec((1, 128), lambda i: (0, 0)),
            pl.BlockSpec((1, D), lambda i: (0, 0)),
            pl.BlockSpec((1, 1), lambda i: (0, 0)),
        ],
        out_specs=[
            pl.BlockSpec((BLK, A), lambda i: (i, 0)),
            pl.BlockSpec((1, BLK), lambda i: (0, i)),
            pl.BlockSpec((BLK, S), lambda i: (i, 0)),
        ],
        out_shape=[
            jax.ShapeDtypeStruct((B, A), jnp.float32),
            jax.ShapeDtypeStruct((1, B), jnp.float32),
            jax.ShapeDtypeStruct((B, S), jnp.float32),
        ],
        compiler_params=pltpu.CompilerParams(
            dimension_semantics=("arbitrary",),
        ),
    )(state, wt, bias, W_value, b_value[None, :])
    std = jnp.zeros((B, A), jnp.float32)
    return (mean, std, value[0], one_hot)


# speedup vs baseline: 1.8054x; 1.0334x over previous
import functools

import jax
import jax.numpy as jnp
from jax import lax
from jax.experimental import pallas as pl
from jax.experimental.pallas import tpu as pltpu

B, D, A, S = 16384, 128, 64, 64
BLK = 8192


def _tc_body(state_ref, wt_ref, bias_ref, wv_ref, bv_ref,
             mean_ref, value_ref, onehot_ref):
    x = state_ref[...]
    res = jnp.dot(x, wt_ref[...]) + bias_ref[...]
    mean_ref[...] = res[:, :A]
    value_ref[...] = lax.dot_general(
        wv_ref[...], x, (((1,), (1,)), ((), ()))) + bv_ref[...]
    logits = res[:, A:]
    idx = jnp.argmax(logits, axis=1)
    onehot_ref[...] = (
        lax.broadcasted_iota(jnp.int32, (BLK, S), 1) == idx[:, None]
    ).astype(jnp.float32)


@jax.jit
def kernel(state, W_skill, b_skill, W_action, b_action, W_value, b_value):
    wt = jnp.concatenate([W_action.T, W_skill.T], axis=1)
    bias = jnp.concatenate([b_action, b_skill])[None, :]

    grid = (B // BLK,)
    mean, value, one_hot = pl.pallas_call(
        _tc_body,
        grid=grid,
        in_specs=[
            pl.BlockSpec((BLK, D), lambda i: (i, 0)),
            pl.BlockSpec((D, 128), lambda i: (0, 0)),
            pl.BlockSpec((1, 128), lambda i: (0, 0)),
            pl.BlockSpec((1, D), lambda i: (0, 0)),
            pl.BlockSpec((1, 1), lambda i: (0, 0)),
        ],
        out_specs=[
            pl.BlockSpec((BLK, A), lambda i: (i, 0)),
            pl.BlockSpec((1, BLK), lambda i: (0, i)),
            pl.BlockSpec((BLK, S), lambda i: (i, 0)),
        ],
        out_shape=[
            jax.ShapeDtypeStruct((B, A), jnp.float32),
            jax.ShapeDtypeStruct((1, B), jnp.float32),
            jax.ShapeDtypeStruct((B, S), jnp.float32),
        ],
        compiler_params=pltpu.CompilerParams(
            dimension_semantics=("parallel",),
        ),
    )(state, wt, bias, W_value, b_value[None, :])
    std = jnp.zeros((B, A), jnp.float32)
    return (mean, std, value[0], one_hot)
